# Initial kernel scaffold; baseline (speedup 1.0000x reference)
import jax
import jax.numpy as jnp
from jax.experimental import pallas as pl


def kernel(feature_view, edge_index, edge_attr, edge_time_emb, boundary_condition, W_msg, b_msg, W_lin, b_lin, ln_gamma, ln_beta):
    raise NotImplementedError("write your pallas kernel here")


# speedup vs baseline: 2.6989x; 2.6989x over previous
import functools

import jax
import jax.numpy as jnp
from jax import lax
from jax.experimental import pallas as pl
from jax.experimental.pallas import tpu as pltpu
from jax.experimental.pallas import tpu_sc as plsc

N = 10000
E = 320000
D = 128
A = 16
T = 16

NC = 2
NS = 16
NW = NC * NS

CH = 128
NCHUNKS = E // CH
ROWS_A = 624
TAIL_ROWS = N - NS * ROWS_A
ZROWS = 16


def _proj_nodes_body(fv_ref, w1_ref, p_ref):
    p_ref[...] = jnp.dot(fv_ref[...], w1_ref[...],
                         preferred_element_type=jnp.float32)


def _proj_edges_body(ea_ref, et_ref, w2a_ref, w2b_ref, b_ref, u_ref):
    u_ref[...] = (
        jnp.dot(ea_ref[...], w2a_ref[...], preferred_element_type=jnp.float32)
        + jnp.dot(et_ref[...], w2b_ref[...], preferred_element_type=jnp.float32)
        + b_ref[...]
    )


def _sc_scatter_body(p_hbm, u_hbm, src_hbm, dst_hbm, out_hbm,
                     src_v, dst_v, rows_v, u_v, zero_v, acc_sh, sem):
    cid = lax.axis_index("c")
    sid = lax.axis_index("s")
    wid = sid * NC + cid

    def zero_buf(i, _):
        r = i // (D // 16)
        c = (i % (D // 16)) * 16
        zero_v[r, pl.ds(c, 16)] = jnp.zeros((16,), jnp.float32)
        return 0
    lax.fori_loop(0, ZROWS * (D // 16), zero_buf, 0, unroll=8)
    row0 = sid * ROWS_A
    for z in range(ROWS_A // ZROWS):
        pltpu.sync_copy(zero_v, acc_sh.at[pl.ds(row0 + z * ZROWS, ZROWS)])

    @pl.when(sid == NS - 1)
    def _zero_tail():
        pltpu.sync_copy(zero_v.at[pl.ds(0, TAIL_ROWS)],
                        acc_sh.at[pl.ds(NS * ROWS_A, TAIL_ROWS)])
    plsc.subcore_barrier()

    my_chunks = NCHUNKS // NW + jnp.where(wid < NCHUNKS % NW, 1, 0)

    def do_chunk(k, _):
        base = (k * NW + wid) * CH
        pltpu.sync_copy(src_hbm.at[pl.ds(base, CH)], src_v)
        pltpu.sync_copy(dst_hbm.at[pl.ds(base, CH)], dst_v)
        pltpu.sync_copy(u_hbm.at[pl.ds(base, CH)], u_v)
        pltpu.async_copy(p_hbm.at[src_v], rows_v, sem).wait()

        def fuse(r, _):
            for c in range(D // 16):
                v = rows_v[r, pl.ds(c * 16, 16)] + u_v[r, pl.ds(c * 16, 16)]
                rows_v[r, pl.ds(c * 16, 16)] = jnp.maximum(v, 0.0)
            return 0
        lax.fori_loop(0, CH, fuse, 0)

        pltpu.sync_copy(rows_v, acc_sh.at[dst_v], add=True)
        return 0
    lax.fori_loop(0, my_chunks, do_chunk, 0)

    plsc.subcore_barrier()

    pltpu.sync_copy(acc_sh.at[pl.ds(row0, ROWS_A)],
                    out_hbm.at[pl.ds(cid * N + row0, ROWS_A)])

    @pl.when(sid == NS - 1)
    def _drain_tail():
        pltpu.sync_copy(acc_sh.at[pl.ds(NS * ROWS_A, TAIL_ROWS)],
                        out_hbm.at[pl.ds(cid * N + NS * ROWS_A, TAIL_ROWS)])


def _final_body(a0_ref, a1_ref, bc_ref, wl_ref, bl_ref, g_ref, be_ref, o_ref):
    h = a0_ref[...] + a1_ref[...] + bc_ref[...]
    y = jnp.dot(h, wl_ref[...], preferred_element_type=jnp.float32) + bl_ref[...]
    mean = jnp.mean(y, axis=-1, keepdims=True)
    var = jnp.mean(jnp.square(y - mean), axis=-1, keepdims=True)
    yn = (y - mean) * lax.rsqrt(var + 1e-5) * g_ref[...] + be_ref[...]
    o_ref[...] = jnp.maximum(yn, 0.0)


def kernel(feature_view, edge_index, edge_attr, edge_time_emb,
           boundary_condition, W_msg, b_msg, W_lin, b_lin, ln_gamma, ln_beta):
    src = edge_index[0]
    dst = edge_index[1]
    w1 = W_msg[:D]
    w2a = W_msg[D:D + A]
    w2b = W_msg[D + A:]
    b2 = b_msg.reshape(1, D)

    BN = 1000
    p = pl.pallas_call(
        _proj_nodes_body,
        grid=(N // BN,),
        in_specs=[
            pl.BlockSpec((BN, D), lambda i: (i, 0)),
            pl.BlockSpec((D, D), lambda i: (0, 0)),
        ],
        out_specs=pl.BlockSpec((BN, D), lambda i: (i, 0)),
        out_shape=jax.ShapeDtypeStruct((N, D), jnp.float32),
    )(feature_view, w1)

    BE = 4000
    u = pl.pallas_call(
        _proj_edges_body,
        grid=(E // BE,),
        in_specs=[
            pl.BlockSpec((BE, A), lambda i: (i, 0)),
            pl.BlockSpec((BE, T), lambda i: (i, 0)),
            pl.BlockSpec((A, D), lambda i: (0, 0)),
            pl.BlockSpec((T, D), lambda i: (0, 0)),
            pl.BlockSpec((1, D), lambda i: (0, 0)),
        ],
        out_specs=pl.BlockSpec((BE, D), lambda i: (i, 0)),
        out_shape=jax.ShapeDtypeStruct((E, D), jnp.float32),
    )(edge_attr, edge_time_emb, w2a, w2b, b2)

    mesh = plsc.VectorSubcoreMesh(core_axis_name="c", subcore_axis_name="s",
                                  num_cores=NC, num_subcores=NS)
    acc2 = pl.kernel(
        _sc_scatter_body,
        out_type=jax.ShapeDtypeStruct((NC * N, D), jnp.float32),
        mesh=mesh,
        scratch_types=[
            pltpu.VMEM((CH,), jnp.int32),
            pltpu.VMEM((CH,), jnp.int32),
            pltpu.VMEM((CH, D), jnp.float32),
            pltpu.VMEM((CH, D), jnp.float32),
            pltpu.VMEM((ZROWS, D), jnp.float32),
            pltpu.VMEM_SHARED((N, D), jnp.float32),
            pltpu.SemaphoreType.DMA,
        ],
    )(p, u, src, dst)

    out = pl.pallas_call(
        _final_body,
        grid=(N // BN,),
        in_specs=[
            pl.BlockSpec((BN, D), lambda i: (i, 0)),
            pl.BlockSpec((BN, D), lambda i: (i + N // BN, 0)),
            pl.BlockSpec((BN, D), lambda i: (i, 0)),
            pl.BlockSpec((D, D), lambda i: (0, 0)),
            pl.BlockSpec((1, D), lambda i: (0, 0)),
            pl.BlockSpec((1, D), lambda i: (0, 0)),
            pl.BlockSpec((1, D), lambda i: (0, 0)),
        ],
        out_specs=pl.BlockSpec((BN, D), lambda i: (i, 0)),
        out_shape=jax.ShapeDtypeStruct((N, D), jnp.float32),
    )(acc2, acc2, boundary_condition, W_lin, b_lin.reshape(1, D),
      ln_gamma.reshape(1, D), ln_beta.reshape(1, D))

    return out
